# baseline (speedup 1.0000x reference)
import functools

import jax
import jax.numpy as jnp
from jax import lax
from jax.experimental import pallas as pl
from jax.experimental.pallas import tpu as pltpu
from jax.experimental.pallas import tpu_sc as plsc

N = 10000
E = 160000
D = 256
H = 128
NC = 2
NS = 16
EPT = E // NS
CHUNK = 48
EPT_PAD = 10560
NROWS = EPT_PAD // (2 * CHUNK)
NSLOT = 5
UNROLL = 4 * NSLOT
NITER = EPT_PAD // CHUNK // UNROLL
ZROWS = 632
NP = NS * ZROWS
OROWS_LAST = N - (NS - 1) * ZROWS

def _dot(a, b):
    return jnp.dot(a, b, preferred_element_type=jnp.float32)


def _ln(h, g, b, eps=1e-5):
    mu = jnp.mean(h, axis=-1, keepdims=True)
    var = jnp.mean((h - mu) * (h - mu), axis=-1, keepdims=True)
    return (h - mu) * jax.lax.rsqrt(var + eps) * g + b


def _unpack(packed_v, row, col, out_ref, out_row, shift, base):
    for q in range(CHUNK // 16):
        v = packed_v[row, pl.ds(col + 16 * q, 16)]
        f = lax.shift_right_logical(v, shift) & 0xFFFF
        out_ref[out_row, pl.ds(16 * q, 16)] = f + base


def _sc_body(hsplit, packed_hbm, zeros_hbm, agg_hbm,
             packed_v, sstage_v, dstage_v,
             rows0_v, rows1_v, rows2_v, rows3_v, rows4_v, acc_sh,
             sem0, sem1, sem2, sem3, sem4):
    c = lax.axis_index("c")
    s = lax.axis_index("s")
    coff = c * N
    pltpu.sync_copy(packed_hbm.at[s], packed_v)

    bufs = [rows0_v, rows1_v, rows2_v, rows3_v, rows4_v]
    sems = [sem0, sem1, sem2, sem3, sem4]

    for i in range(NSLOT):
        _unpack(packed_v, i // 2, (i % 2) * CHUNK, sstage_v, i, 0, coff)
        pltpu.async_copy(hsplit.at[sstage_v.at[i, pl.ds(0, CHUNK)]],
                         bufs[i], sems[i])

    pltpu.sync_copy(zeros_hbm.at[pl.ds(s * ZROWS, ZROWS)],
                    acc_sh.at[pl.ds(s * ZROWS, ZROWS)])
    plsc.subcore_barrier()

    def step(t, carry):
        r0 = (UNROLL // 2) * t
        for i in range(UNROLL):
            b = i % NSLOT
            row = r0 + i // 2
            col = (i % 2) * CHUNK
            nrow = r0 + (i + NSLOT) // 2
            ncol = ((i + NSLOT) % 2) * CHUNK
            pltpu.make_async_copy(hsplit.at[sstage_v.at[b, pl.ds(0, CHUNK)]],
                                  bufs[b], sems[b]).wait()
            _unpack(packed_v, row, col, dstage_v, 0, 16, 0)
            pltpu.sync_copy(bufs[b], acc_sh.at[dstage_v.at[0, pl.ds(0, CHUNK)]],
                            add=True)

            if i < UNROLL - NSLOT:
                _unpack(packed_v, nrow, ncol, sstage_v, b, 0, coff)
                pltpu.async_copy(hsplit.at[sstage_v.at[b, pl.ds(0, CHUNK)]],
                                 bufs[b], sems[b])
            else:
                @pl.when(t + 1 < NITER)
                def _():
                    _unpack(packed_v, nrow, ncol, sstage_v, b, 0, coff)
                    pltpu.async_copy(hsplit.at[sstage_v.at[b, pl.ds(0, CHUNK)]],
                                     bufs[b], sems[b])

        return carry

    lax.fori_loop(0, NITER, step, 0)
    plsc.subcore_barrier()

    @pl.when(s < NS - 1)
    def _():
        pltpu.sync_copy(acc_sh.at[pl.ds(s * ZROWS, ZROWS)],
                        agg_hbm.at[pl.ds(c * N + s * ZROWS, ZROWS)])

    @pl.when(s == NS - 1)
    def _():
        pltpu.sync_copy(acc_sh.at[pl.ds((NS - 1) * ZROWS, OROWS_LAST)],
                        agg_hbm.at[pl.ds(c * N + (NS - 1) * ZROWS, OROWS_LAST)])


_sc_msg = functools.partial(
    pl.kernel,
    out_type=jax.ShapeDtypeStruct((NC * N, H), jnp.float32),
    mesh=plsc.VectorSubcoreMesh(core_axis_name="c", subcore_axis_name="s",
                                num_cores=NC, num_subcores=NS),
    scratch_types=[
        pltpu.VMEM((NROWS, 128), jnp.int32),
        pltpu.VMEM((NSLOT, 128), jnp.int32),
        pltpu.VMEM((1, 128), jnp.int32),
        pltpu.VMEM((CHUNK, H), jnp.float32),
        pltpu.VMEM((CHUNK, H), jnp.float32),
        pltpu.VMEM((CHUNK, H), jnp.float32),
        pltpu.VMEM((CHUNK, H), jnp.float32),
        pltpu.VMEM((CHUNK, H), jnp.float32),
        pltpu.VMEM_SHARED((NP, H), jnp.float32),
    ] + [pltpu.SemaphoreType.DMA] * NSLOT,
)(_sc_body)


_BLK = 2000
_GRID = N // _BLK


_W_SPEC = pl.BlockSpec((D, D), lambda i: (0, 0))
_V_SPEC = pl.BlockSpec((1, D), lambda i: (0, 0))
_HS_SPEC = pl.BlockSpec((NC, _BLK, H), lambda i: (0, i, 0))
_HS_TYPE = jax.ShapeDtypeStruct((NC, N, H), jnp.float32)


def _split(o_ref, z):
    o_ref[0] = z[:, :H]
    o_ref[1] = z[:, H:]


def _pre_body(x_ref, w_ref, b_ref, g_ref, bt_ref, rootw_ref, rb_ref,
              ohs_ref, ort_ref):
    h = _dot(x_ref[...], w_ref[...]) + b_ref[...]
    h = _ln(h, g_ref[...], bt_ref[...])
    h = jnp.maximum(h, 0.0)
    _split(ohs_ref, h)
    _split(ort_ref, _dot(h, rootw_ref[...]) + rb_ref[...])


def _pre(x, w, b, g, bt, rootw, rb):
    return pl.pallas_call(
        _pre_body,
        grid=(_GRID,),
        in_specs=[
            pl.BlockSpec((_BLK, D), lambda i: (i, 0)),
            _W_SPEC, _V_SPEC, _V_SPEC, _V_SPEC, _W_SPEC, _V_SPEC,
        ],
        out_specs=(_HS_SPEC, _HS_SPEC),
        out_shape=(_HS_TYPE, _HS_TYPE),
    )(x, w, b, g, bt, rootw, rb)


def _mix(agg_ref, rt_ref, rw_ref, g_ref, bt_ref):
    rt = jnp.concatenate([rt_ref[0], rt_ref[1]], axis=1)
    rw = rw_ref[...]
    z = _dot(agg_ref[0], rw[:H]) + _dot(agg_ref[1], rw[H:]) + rt
    z = jnp.maximum(z, 0.0)
    return _ln(z, g_ref[...], bt_ref[...])


def _layer_body(agg_ref, rt_ref, rw_ref, g_ref, bt_ref, rootw_ref, rb_ref,
                ohs_ref, ort_ref):
    z = _mix(agg_ref, rt_ref, rw_ref, g_ref, bt_ref)
    _split(ohs_ref, z)
    _split(ort_ref, _dot(z, rootw_ref[...]) + rb_ref[...])


def _layer(agg, rt, rw, g, bt, rootw, rb):
    return pl.pallas_call(
        _layer_body,
        grid=(_GRID,),
        in_specs=[_HS_SPEC, _HS_SPEC, _W_SPEC, _V_SPEC, _V_SPEC,
                  _W_SPEC, _V_SPEC],
        out_specs=(_HS_SPEC, _HS_SPEC),
        out_shape=(_HS_TYPE, _HS_TYPE),
    )(agg, rt, rw, g, bt, rootw, rb)


def _final_body(agg_ref, rt_ref, rw_ref, g_ref, bt_ref, x_ref, lw_ref,
                lb_ref, o_ref):
    z = _mix(agg_ref, rt_ref, rw_ref, g_ref, bt_ref)
    y = _dot(z, lw_ref[...]) + lb_ref[...]
    o_ref[...] = jnp.concatenate([x_ref[...], y], axis=1)


def _final(agg, rt, rw, g, bt, x, lw, lb):
    d_out = D + lw.shape[1]
    return pl.pallas_call(
        _final_body,
        grid=(_GRID,),
        in_specs=[
            _HS_SPEC, _HS_SPEC, _W_SPEC, _V_SPEC, _V_SPEC,
            pl.BlockSpec((_BLK, D), lambda i: (i, 0)),
            pl.BlockSpec((D, lw.shape[1]), lambda i: (0, 0)),
            pl.BlockSpec((1, lw.shape[1]), lambda i: (0, 0)),
        ],
        out_specs=pl.BlockSpec((_BLK, d_out), lambda i: (i, 0)),
        out_shape=jax.ShapeDtypeStruct((N, d_out), jnp.float32),
    )(agg, rt, rw, g, bt, x, lw, lb)


def kernel(x, edge_index, pre_W, pre_b, pre_ln_g, pre_ln_b,
           rel_W, rel_b, root_W, ln_g, ln_b, lin_W, lin_b):
    src = edge_index[0]
    dst = edge_index[1]
    srcp = jnp.pad(src.reshape(NS, EPT), ((0, 0), (0, EPT_PAD - EPT)))
    dstp = jnp.pad(dst.reshape(NS, EPT), ((0, 0), (0, EPT_PAD - EPT)),
                   constant_values=N)
    packed = (srcp.astype(jnp.int32) | (dstp.astype(jnp.int32) << 16))
    packed = packed.reshape(NS, NROWS, 2 * CHUNK)
    packed = jnp.pad(packed, ((0, 0), (0, 0), (0, 128 - 2 * CHUNK)))
    zeros = jnp.zeros((NP, H), jnp.float32)

    nl = rel_W.shape[0]
    hs, rt = _pre(x, pre_W, pre_b.reshape(1, D), pre_ln_g.reshape(1, D),
                  pre_ln_b.reshape(1, D), root_W[0], rel_b[0].reshape(1, D))
    for i in range(nl - 1):
        agg = _sc_msg(hs.reshape(NC * N, H), packed, zeros)
        hs, rt = _layer(agg.reshape(NC, N, H), rt, rel_W[i],
                        ln_g[i].reshape(1, D), ln_b[i].reshape(1, D),
                        root_W[i + 1], rel_b[i + 1].reshape(1, D))
    agg = _sc_msg(hs.reshape(NC * N, H), packed, zeros)
    return _final(agg.reshape(NC, N, H), rt, rel_W[nl - 1],
                  ln_g[nl - 1].reshape(1, D), ln_b[nl - 1].reshape(1, D),
                  x, lin_W, lin_b.reshape(1, lin_W.shape[1]))


# speedup vs baseline: 2.6861x; 2.6861x over previous
import functools

import jax
import jax.numpy as jnp
from jax import lax
from jax.experimental import pallas as pl
from jax.experimental.pallas import tpu as pltpu
from jax.experimental.pallas import tpu_sc as plsc

N = 10000
E = 160000
D = 256
H = 128
NC = 2
NS = 16
EPT = E // NS
CHUNK = 48
EPT_PAD = 10080
NROWS = EPT_PAD // (2 * CHUNK)
NSLOT = 5
NITER = EPT_PAD // CHUNK // (2 * NSLOT)
ZROWS = 632
NP = NS * ZROWS
OROWS_LAST = N - (NS - 1) * ZROWS

def _dot(a, b):
    return jnp.dot(a, b, preferred_element_type=jnp.float32)


def _ln(h, g, b, eps=1e-5):
    mu = jnp.mean(h, axis=-1, keepdims=True)
    var = jnp.mean((h - mu) * (h - mu), axis=-1, keepdims=True)
    return (h - mu) * jax.lax.rsqrt(var + eps) * g + b


def _unpack(packed_v, row, col, out_ref, out_row, shift, base):
    for q in range(CHUNK // 16):
        v = packed_v[row, pl.ds(col + 16 * q, 16)]
        f = lax.shift_right_logical(v, shift) & 0xFFFF
        out_ref[out_row, pl.ds(16 * q, 16)] = f + base


def _sc_body(hsplit, packed_hbm, zeros_hbm, agg_hbm,
             packed_v, sstage_v, dstage_v,
             rows0_v, rows1_v, rows2_v, rows3_v, rows4_v, acc_sh,
             sem0, sem1, sem2, sem3, sem4):
    c = lax.axis_index("c")
    s = lax.axis_index("s")
    coff = c * N
    pltpu.sync_copy(packed_hbm.at[s], packed_v)

    bufs = [rows0_v, rows1_v, rows2_v, rows3_v, rows4_v]
    sems = [sem0, sem1, sem2, sem3, sem4]

    for i in range(NSLOT):
        _unpack(packed_v, i // 2, (i % 2) * CHUNK, sstage_v, i, 0, coff)
        pltpu.async_copy(hsplit.at[sstage_v.at[i, pl.ds(0, CHUNK)]],
                         bufs[i], sems[i])

    pltpu.sync_copy(zeros_hbm.at[pl.ds(s * ZROWS, ZROWS)],
                    acc_sh.at[pl.ds(s * ZROWS, ZROWS)])
    plsc.subcore_barrier()

    def step(t, carry):
        r0 = NSLOT * t
        for i in range(2 * NSLOT):
            b = i % NSLOT
            row = r0 + i // 2
            col = (i % 2) * CHUNK
            nrow = r0 + (i + NSLOT) // 2
            ncol = ((i + NSLOT) % 2) * CHUNK
            pltpu.make_async_copy(hsplit.at[sstage_v.at[b, pl.ds(0, CHUNK)]],
                                  bufs[b], sems[b]).wait()
            _unpack(packed_v, row, col, dstage_v, 0, 16, 0)
            pltpu.sync_copy(bufs[b], acc_sh.at[dstage_v.at[0, pl.ds(0, CHUNK)]],
                            add=True)

            if i < NSLOT:
                _unpack(packed_v, nrow, ncol, sstage_v, b, 0, coff)
                pltpu.async_copy(hsplit.at[sstage_v.at[b, pl.ds(0, CHUNK)]],
                                 bufs[b], sems[b])
            else:
                @pl.when(t + 1 < NITER)
                def _():
                    _unpack(packed_v, nrow, ncol, sstage_v, b, 0, coff)
                    pltpu.async_copy(hsplit.at[sstage_v.at[b, pl.ds(0, CHUNK)]],
                                     bufs[b], sems[b])

        return carry

    lax.fori_loop(0, NITER, step, 0)
    plsc.subcore_barrier()

    @pl.when(s < NS - 1)
    def _():
        pltpu.sync_copy(acc_sh.at[pl.ds(s * ZROWS, ZROWS)],
                        agg_hbm.at[pl.ds(c * N + s * ZROWS, ZROWS)])

    @pl.when(s == NS - 1)
    def _():
        pltpu.sync_copy(acc_sh.at[pl.ds((NS - 1) * ZROWS, OROWS_LAST)],
                        agg_hbm.at[pl.ds(c * N + (NS - 1) * ZROWS, OROWS_LAST)])


_sc_msg = functools.partial(
    pl.kernel,
    out_type=jax.ShapeDtypeStruct((NC * N, H), jnp.float32),
    mesh=plsc.VectorSubcoreMesh(core_axis_name="c", subcore_axis_name="s",
                                num_cores=NC, num_subcores=NS),
    scratch_types=[
        pltpu.VMEM((NROWS, 128), jnp.int32),
        pltpu.VMEM((NSLOT, 128), jnp.int32),
        pltpu.VMEM((1, 128), jnp.int32),
        pltpu.VMEM((CHUNK, H), jnp.float32),
        pltpu.VMEM((CHUNK, H), jnp.float32),
        pltpu.VMEM((CHUNK, H), jnp.float32),
        pltpu.VMEM((CHUNK, H), jnp.float32),
        pltpu.VMEM((CHUNK, H), jnp.float32),
        pltpu.VMEM_SHARED((NP, H), jnp.float32),
    ] + [pltpu.SemaphoreType.DMA] * NSLOT,
)(_sc_body)


_BLK = 2000
_GRID = N // _BLK


_W_SPEC = pl.BlockSpec((D, D), lambda i: (0, 0))
_V_SPEC = pl.BlockSpec((1, D), lambda i: (0, 0))
_HS_SPEC = pl.BlockSpec((NC, _BLK, H), lambda i: (0, i, 0))
_HS_TYPE = jax.ShapeDtypeStruct((NC, N, H), jnp.float32)


def _split(o_ref, z):
    o_ref[0] = z[:, :H]
    o_ref[1] = z[:, H:]


def _pre_body(x_ref, w_ref, b_ref, g_ref, bt_ref, rootw_ref, rb_ref,
              ohs_ref, ort_ref):
    h = _dot(x_ref[...], w_ref[...]) + b_ref[...]
    h = _ln(h, g_ref[...], bt_ref[...])
    h = jnp.maximum(h, 0.0)
    _split(ohs_ref, h)
    _split(ort_ref, _dot(h, rootw_ref[...]) + rb_ref[...])


def _pre(x, w, b, g, bt, rootw, rb):
    return pl.pallas_call(
        _pre_body,
        grid=(_GRID,),
        in_specs=[
            pl.BlockSpec((_BLK, D), lambda i: (i, 0)),
            _W_SPEC, _V_SPEC, _V_SPEC, _V_SPEC, _W_SPEC, _V_SPEC,
        ],
        out_specs=(_HS_SPEC, _HS_SPEC),
        out_shape=(_HS_TYPE, _HS_TYPE),
    )(x, w, b, g, bt, rootw, rb)


def _mix(agg_ref, rt_ref, rw_ref, g_ref, bt_ref):
    rt = jnp.concatenate([rt_ref[0], rt_ref[1]], axis=1)
    rw = rw_ref[...]
    z = _dot(agg_ref[0], rw[:H]) + _dot(agg_ref[1], rw[H:]) + rt
    z = jnp.maximum(z, 0.0)
    return _ln(z, g_ref[...], bt_ref[...])


def _layer_body(agg_ref, rt_ref, rw_ref, g_ref, bt_ref, rootw_ref, rb_ref,
                ohs_ref, ort_ref):
    z = _mix(agg_ref, rt_ref, rw_ref, g_ref, bt_ref)
    _split(ohs_ref, z)
    _split(ort_ref, _dot(z, rootw_ref[...]) + rb_ref[...])


def _layer(agg, rt, rw, g, bt, rootw, rb):
    return pl.pallas_call(
        _layer_body,
        grid=(_GRID,),
        in_specs=[_HS_SPEC, _HS_SPEC, _W_SPEC, _V_SPEC, _V_SPEC,
                  _W_SPEC, _V_SPEC],
        out_specs=(_HS_SPEC, _HS_SPEC),
        out_shape=(_HS_TYPE, _HS_TYPE),
    )(agg, rt, rw, g, bt, rootw, rb)


def _final_body(agg_ref, rt_ref, rw_ref, g_ref, bt_ref, x_ref, lw_ref,
                lb_ref, o_ref):
    z = _mix(agg_ref, rt_ref, rw_ref, g_ref, bt_ref)
    y = _dot(z, lw_ref[...]) + lb_ref[...]
    o_ref[...] = jnp.concatenate([x_ref[...], y], axis=1)


def _final(agg, rt, rw, g, bt, x, lw, lb):
    d_out = D + lw.shape[1]
    return pl.pallas_call(
        _final_body,
        grid=(_GRID,),
        in_specs=[
            _HS_SPEC, _HS_SPEC, _W_SPEC, _V_SPEC, _V_SPEC,
            pl.BlockSpec((_BLK, D), lambda i: (i, 0)),
            pl.BlockSpec((D, lw.shape[1]), lambda i: (0, 0)),
            pl.BlockSpec((1, lw.shape[1]), lambda i: (0, 0)),
        ],
        out_specs=pl.BlockSpec((_BLK, d_out), lambda i: (i, 0)),
        out_shape=jax.ShapeDtypeStruct((N, d_out), jnp.float32),
    )(agg, rt, rw, g, bt, x, lw, lb)


def kernel(x, edge_index, pre_W, pre_b, pre_ln_g, pre_ln_b,
           rel_W, rel_b, root_W, ln_g, ln_b, lin_W, lin_b):
    src = edge_index[0]
    dst = edge_index[1]
    srcp = jnp.pad(src.reshape(NS, EPT), ((0, 0), (0, EPT_PAD - EPT)))
    dstp = jnp.pad(dst.reshape(NS, EPT), ((0, 0), (0, EPT_PAD - EPT)),
                   constant_values=N)
    packed = (srcp.astype(jnp.int32) | (dstp.astype(jnp.int32) << 16))
    packed = packed.reshape(NS, NROWS, 2 * CHUNK)
    packed = jnp.pad(packed, ((0, 0), (0, 0), (0, 128 - 2 * CHUNK)))
    zeros = jnp.zeros((NP, H), jnp.float32)

    nl = rel_W.shape[0]
    hs, rt = _pre(x, pre_W, pre_b.reshape(1, D), pre_ln_g.reshape(1, D),
                  pre_ln_b.reshape(1, D), root_W[0], rel_b[0].reshape(1, D))
    for i in range(nl - 1):
        agg = _sc_msg(hs.reshape(NC * N, H), packed, zeros)
        hs, rt = _layer(agg.reshape(NC, N, H), rt, rel_W[i],
                        ln_g[i].reshape(1, D), ln_b[i].reshape(1, D),
                        root_W[i + 1], rel_b[i + 1].reshape(1, D))
    agg = _sc_msg(hs.reshape(NC * N, H), packed, zeros)
    return _final(agg.reshape(NC, N, H), rt, rel_W[nl - 1],
                  ln_g[nl - 1].reshape(1, D), ln_b[nl - 1].reshape(1, D),
                  x, lin_W, lin_b.reshape(1, lin_W.shape[1]))
